# baseline (speedup 1.0000x reference)
import functools

import jax
import jax.numpy as jnp
from jax import lax
from jax.experimental import pallas as pl
from jax.experimental.pallas import tpu as pltpu
from jax.experimental.pallas import tpu_sc as plsc

BATCH = 16384
EMB_DIM = 64
NUM_CLASSES = 100000
NUM_CORES = 2
NUM_SUBCORES = 16
NUM_WORKERS = NUM_CORES * NUM_SUBCORES
DIMS_PER_WORKER = EMB_DIM // NUM_WORKERS
FCHUNK = 2048
NUM_FCHUNKS = BATCH // FCHUNK
LANES = 16
VECS_PER_FCHUNK = FCHUNK // LANES


def _body(fT_hbm, lab_hbm, cT_hbm, out_hbm, row_v, lab_v, fbuf0, fbuf1, acc_v,
          rsem, lsem, fsem):
    fbufs = (fbuf0, fbuf1)
    wid = lax.axis_index("s") * NUM_CORES + lax.axis_index("c")
    d0 = wid * DIMS_PER_WORKER

    row_copies = [pltpu.async_copy(cT_hbm.at[d0], row_v, rsem)]
    feat_copies = [
        pltpu.async_copy(fT_hbm.at[d0, pl.ds(0, FCHUNK)], fbuf0, fsem)
    ]
    lcopy = pltpu.async_copy(lab_hbm, lab_v, lsem)
    lcopy.wait()

    acc = jnp.zeros((LANES,), jnp.float32)
    for k in range(DIMS_PER_WORKER):
        d = d0 + k
        for rc in row_copies:
            rc.wait()
        for c in range(NUM_FCHUNKS):
            g = k * NUM_FCHUNKS + c
            if c + 1 < NUM_FCHUNKS:
                feat_copies.append(
                    pltpu.async_copy(
                        fT_hbm.at[d, pl.ds((c + 1) * FCHUNK, FCHUNK)],
                        fbufs[(g + 1) % 2], fsem))
            elif k + 1 < DIMS_PER_WORKER:
                feat_copies.append(
                    pltpu.async_copy(
                        fT_hbm.at[d + 1, pl.ds(0, FCHUNK)],
                        fbufs[(g + 1) % 2], fsem))
            feat_copies[g].wait()
            fb = fbufs[g % 2]
            base = c * FCHUNK

            def chunk(i, acc, fb=fb, base=base):
                idx = lab_v[pl.ds(base + i * LANES, LANES)]
                cv = plsc.load_gather(row_v, [idx])
                fv = fb[pl.ds(i * LANES, LANES)]
                dv = fv - cv
                return acc + dv * dv

            acc = lax.fori_loop(0, VECS_PER_FCHUNK, chunk, acc, unroll=2)
        if k + 1 < DIMS_PER_WORKER:
            row_copies = [pltpu.async_copy(cT_hbm.at[d + 1], row_v, rsem)]

    acc_v[...] = acc
    pltpu.sync_copy(acc_v, out_hbm.at[wid])


@jax.jit
def _center_loss(features, labels, centers):
    fT = features.T
    cT = centers.T
    lab = labels.astype(jnp.int32)
    mesh = plsc.VectorSubcoreMesh(core_axis_name="c", subcore_axis_name="s")
    partials = pl.kernel(
        _body,
        out_type=jax.ShapeDtypeStruct((NUM_WORKERS, LANES), jnp.float32),
        mesh=mesh,
        scratch_types=[
            pltpu.VMEM((NUM_CLASSES,), jnp.float32),
            pltpu.VMEM((BATCH,), jnp.int32),
            pltpu.VMEM((FCHUNK,), jnp.float32),
            pltpu.VMEM((FCHUNK,), jnp.float32),
            pltpu.VMEM((LANES,), jnp.float32),
            pltpu.SemaphoreType.DMA,
            pltpu.SemaphoreType.DMA,
            pltpu.SemaphoreType.DMA,
        ],
        compiler_params=pltpu.CompilerParams(needs_layout_passes=False),
    )(fT, lab, cT)
    return jnp.sum(partials) / (2.0 * features.shape[0])


def kernel(features, labels, centers):
    return _center_loss(features, labels, centers)


# speedup vs baseline: 1.0805x; 1.0805x over previous
import functools

import jax
import jax.numpy as jnp
from jax import lax
from jax.experimental import pallas as pl
from jax.experimental.pallas import tpu as pltpu
from jax.experimental.pallas import tpu_sc as plsc

BATCH = 16384
EMB_DIM = 64
NUM_CLASSES = 100000
NUM_CORES = 2
NUM_SUBCORES = 16
NUM_WORKERS = NUM_CORES * NUM_SUBCORES
DIMS_PER_WORKER = EMB_DIM // NUM_WORKERS
FCHUNK = 4096
NUM_FCHUNKS = BATCH // FCHUNK
LANES = 16
VECS_PER_FCHUNK = FCHUNK // LANES


def _body(fT_hbm, lab_hbm, cT_hbm, out_hbm, row_v, lab_v, fbuf0, fbuf1, acc_v,
          rsem, lsem, fsem):
    fbufs = (fbuf0, fbuf1)
    wid = lax.axis_index("s") * NUM_CORES + lax.axis_index("c")
    d0 = wid * DIMS_PER_WORKER

    row_copies = [pltpu.async_copy(cT_hbm.at[d0], row_v, rsem)]
    feat_copies = [
        pltpu.async_copy(fT_hbm.at[d0, pl.ds(0, FCHUNK)], fbuf0, fsem)
    ]
    lcopy = pltpu.async_copy(lab_hbm, lab_v, lsem)
    lcopy.wait()

    acc = jnp.zeros((LANES,), jnp.float32)
    for k in range(DIMS_PER_WORKER):
        d = d0 + k
        for rc in row_copies:
            rc.wait()
        for c in range(NUM_FCHUNKS):
            g = k * NUM_FCHUNKS + c
            if c + 1 < NUM_FCHUNKS:
                feat_copies.append(
                    pltpu.async_copy(
                        fT_hbm.at[d, pl.ds((c + 1) * FCHUNK, FCHUNK)],
                        fbufs[(g + 1) % 2], fsem))
            elif k + 1 < DIMS_PER_WORKER:
                feat_copies.append(
                    pltpu.async_copy(
                        fT_hbm.at[d + 1, pl.ds(0, FCHUNK)],
                        fbufs[(g + 1) % 2], fsem))
            feat_copies[g].wait()
            fb = fbufs[g % 2]
            base = c * FCHUNK

            def chunk(i, acc, fb=fb, base=base):
                idx = lab_v[pl.ds(base + i * LANES, LANES)]
                cv = plsc.load_gather(row_v, [idx])
                fv = fb[pl.ds(i * LANES, LANES)]
                dv = fv - cv
                return acc + dv * dv

            acc = lax.fori_loop(0, VECS_PER_FCHUNK, chunk, acc, unroll=4)
        if k + 1 < DIMS_PER_WORKER:
            row_copies = [pltpu.async_copy(cT_hbm.at[d + 1], row_v, rsem)]

    acc_v[...] = acc
    pltpu.sync_copy(acc_v, out_hbm.at[wid])


@jax.jit
def _center_loss(features, labels, centers):
    fT = features.T
    cT = centers.T
    lab = labels.astype(jnp.int32)
    mesh = plsc.VectorSubcoreMesh(core_axis_name="c", subcore_axis_name="s")
    partials = pl.kernel(
        _body,
        out_type=jax.ShapeDtypeStruct((NUM_WORKERS, LANES), jnp.float32),
        mesh=mesh,
        scratch_types=[
            pltpu.VMEM((NUM_CLASSES,), jnp.float32),
            pltpu.VMEM((BATCH,), jnp.int32),
            pltpu.VMEM((FCHUNK,), jnp.float32),
            pltpu.VMEM((FCHUNK,), jnp.float32),
            pltpu.VMEM((LANES,), jnp.float32),
            pltpu.SemaphoreType.DMA,
            pltpu.SemaphoreType.DMA,
            pltpu.SemaphoreType.DMA,
        ],
        compiler_params=pltpu.CompilerParams(needs_layout_passes=False),
    )(fT, lab, cT)
    return jnp.sum(partials) / (2.0 * features.shape[0])


def kernel(features, labels, centers):
    return _center_loss(features, labels, centers)


# speedup vs baseline: 1.0815x; 1.0009x over previous
import functools

import jax
import jax.numpy as jnp
from jax import lax
from jax.experimental import pallas as pl
from jax.experimental.pallas import tpu as pltpu
from jax.experimental.pallas import tpu_sc as plsc

BATCH = 16384
EMB_DIM = 64
NUM_CLASSES = 100000
NUM_CORES = 2
NUM_SUBCORES = 16
NUM_WORKERS = NUM_CORES * NUM_SUBCORES
DIMS_PER_WORKER = EMB_DIM // NUM_WORKERS
FCHUNK = 4096
NUM_FCHUNKS = BATCH // FCHUNK
LANES = 16
VECS_PER_FCHUNK = FCHUNK // LANES


def _body(fT_hbm, lab_hbm, cT_hbm, out_hbm, row_v, lab_v, fbuf0, fbuf1, acc_v,
          rsem, lsem, fsem):
    fbufs = (fbuf0, fbuf1)
    wid = lax.axis_index("s") * NUM_CORES + lax.axis_index("c")
    d0 = wid * DIMS_PER_WORKER

    row_copies = [pltpu.async_copy(cT_hbm.at[d0], row_v, rsem)]
    feat_copies = [
        pltpu.async_copy(fT_hbm.at[d0, pl.ds(0, FCHUNK)], fbuf0, fsem)
    ]
    lcopy = pltpu.async_copy(lab_hbm, lab_v, lsem)
    lcopy.wait()

    acc = jnp.zeros((LANES,), jnp.float32)
    for k in range(DIMS_PER_WORKER):
        d = d0 + k
        for rc in row_copies:
            rc.wait()
        for c in range(NUM_FCHUNKS):
            g = k * NUM_FCHUNKS + c
            if c + 1 < NUM_FCHUNKS:
                feat_copies.append(
                    pltpu.async_copy(
                        fT_hbm.at[d, pl.ds((c + 1) * FCHUNK, FCHUNK)],
                        fbufs[(g + 1) % 2], fsem))
            elif k + 1 < DIMS_PER_WORKER:
                feat_copies.append(
                    pltpu.async_copy(
                        fT_hbm.at[d + 1, pl.ds(0, FCHUNK)],
                        fbufs[(g + 1) % 2], fsem))
            feat_copies[g].wait()
            fb = fbufs[g % 2]
            base = c * FCHUNK

            def chunk(i, acc, fb=fb, base=base):
                idx = lab_v[pl.ds(base + i * LANES, LANES)]
                cv = plsc.load_gather(row_v, [idx])
                fv = fb[pl.ds(i * LANES, LANES)]
                dv = fv - cv
                return acc + dv * dv

            acc = lax.fori_loop(0, VECS_PER_FCHUNK, chunk, acc, unroll=8)
        if k + 1 < DIMS_PER_WORKER:
            row_copies = [pltpu.async_copy(cT_hbm.at[d + 1], row_v, rsem)]

    acc_v[...] = acc
    pltpu.sync_copy(acc_v, out_hbm.at[wid])


@jax.jit
def _center_loss(features, labels, centers):
    fT = features.T
    cT = centers.T
    lab = labels.astype(jnp.int32)
    mesh = plsc.VectorSubcoreMesh(core_axis_name="c", subcore_axis_name="s")
    partials = pl.kernel(
        _body,
        out_type=jax.ShapeDtypeStruct((NUM_WORKERS, LANES), jnp.float32),
        mesh=mesh,
        scratch_types=[
            pltpu.VMEM((NUM_CLASSES,), jnp.float32),
            pltpu.VMEM((BATCH,), jnp.int32),
            pltpu.VMEM((FCHUNK,), jnp.float32),
            pltpu.VMEM((FCHUNK,), jnp.float32),
            pltpu.VMEM((LANES,), jnp.float32),
            pltpu.SemaphoreType.DMA,
            pltpu.SemaphoreType.DMA,
            pltpu.SemaphoreType.DMA,
        ],
        compiler_params=pltpu.CompilerParams(needs_layout_passes=False),
    )(fT, lab, cT)
    return jnp.sum(partials) / (2.0 * features.shape[0])


def kernel(features, labels, centers):
    return _center_loss(features, labels, centers)
